# baseline (speedup 1.0000x reference)
import jax
import jax.numpy as jnp
from jax import lax
from jax.experimental import pallas as pl
from jax.experimental.pallas import tpu as pltpu

_BR = 512


def _bdot(a, b, dims=((1,), (0,))):
    return lax.dot_general(a.astype(jnp.bfloat16), b.astype(jnp.bfloat16),
                           (dims, ((), ())),
                           preferred_element_type=jnp.float32)


def _colsum_body(A_ref, d_ref, acc):
    r = pl.program_id(0)
    BR, N = A_ref.shape

    @pl.when(r == 0)
    def _():
        acc[...] = jnp.zeros_like(acc)

    blk = A_ref[...]
    a = acc[...]
    col_ids = lax.broadcasted_iota(jnp.int32, (8, N), 1)
    base_ids = lax.broadcasted_iota(jnp.int32, (8, N), 0) + r * BR
    for i in range(BR // 8):
        eye = jnp.where(base_ids + 8 * i == col_ids, 1.0, 0.0)
        a = a + (blk[8 * i:8 * i + 8] + eye)
    acc[...] = a

    @pl.when(r == pl.num_programs(0) - 1)
    def _():
        t = a[0:4] + a[4:8]
        t2 = t[0:2] + t[2:4]
        cs = t2[0:1] + t2[1:2]
        d_ref[...] = (cs + 1e-5) ** -0.5


def _layer1_body(A_ref, dcol_ref, drow_ref, x_ref, W0_ref, Lb_ref, h_ref,
                 Hpre, xb):
    r = pl.program_id(0)
    BR, N = A_ref.shape

    @pl.when(r == 0)
    def _():
        xb[...] = x_ref[...].astype(jnp.bfloat16)

    row_ids = lax.broadcasted_iota(jnp.int32, (BR, N), 0) + r * BR
    col_ids = lax.broadcasted_iota(jnp.int32, (BR, N), 1)
    A_hat = A_ref[...] + jnp.where(row_ids == col_ids, 1.0, 0.0)
    Lblk = ((dcol_ref[...] * A_hat) * drow_ref[...]).astype(jnp.bfloat16)
    Lb_ref[...] = Lblk
    Hpre[pl.ds(r * BR, BR), :] = lax.dot_general(
        Lblk, xb[...], ((((1,), (0,))), ((), ())),
        preferred_element_type=jnp.float32)

    @pl.when(r == pl.num_programs(0) - 1)
    def _():
        h_ref[...] = _bdot(Hpre[...], W0_ref[...])


def _layerN_body(Lb_ref, xk_ref, W_ref, h_ref, Hpre, yb):
    r = pl.program_id(0)
    BR = Lb_ref.shape[0]

    @pl.when(r == 0)
    def _():
        yb[...] = xk_ref[...].astype(jnp.bfloat16)

    Hpre[pl.ds(r * BR, BR), :] = lax.dot_general(
        Lb_ref[...], yb[...], ((((1,), (0,))), ((), ())),
        preferred_element_type=jnp.float32)

    @pl.when(r == pl.num_programs(0) - 1)
    def _():
        h_ref[...] = _bdot(Hpre[...], W_ref[...])


def _decoder_body(x1_ref, x2_ref, x3_ref, decW_ref, decb_ref, seqW_ref,
                  seqb_ref, cmap_ref, seq_ref, hdec):
    r = pl.program_id(0)
    BR = cmap_ref.shape[0]
    F = x1_ref.shape[1]

    @pl.when(r == 0)
    def _():
        x1 = x1_ref[...]
        x2 = x2_ref[...]
        x3 = x3_ref[...]
        dW = decW_ref[...]
        hdec[...] = (_bdot(x1, dW[0:F]) + _bdot(x2, dW[F:2 * F])
                     + _bdot(x3, dW[2 * F:3 * F]) + decb_ref[...])
        sW = seqW_ref[...]
        s = (_bdot(x1, sW[0:F]) + _bdot(x2, sW[F:2 * F])
             + _bdot(x3, sW[2 * F:3 * F]) + seqb_ref[...])
        m = jnp.max(s, axis=-1, keepdims=True)
        lse = jnp.log(jnp.sum(jnp.exp(s - m), axis=-1, keepdims=True))
        seq_ref[...] = s - m - lse

    hb = hdec[pl.ds(r * BR, BR), :]
    logits = _bdot(hb, hdec[...], ((1,), (1,)))
    cmap_ref[...] = jax.nn.sigmoid(logits)


def kernel(adj, x, W0, g0, beta0, W1, g1, beta1, W2, g2, beta2,
           dec_W, dec_b, seq_W, seq_b):
    Bb, N, _ = adj.shape
    A = adj.reshape(N, N)
    xf = x.reshape(N, x.shape[-1])
    F = W0.shape[1]
    S = seq_W.shape[1]
    FD = dec_W.shape[1]
    R = N // _BR
    f32 = jnp.float32
    bf16 = jnp.bfloat16
    cparams = pltpu.CompilerParams(
        dimension_semantics=("arbitrary",),
        vmem_limit_bytes=int(63.9 * 1024 * 1024))

    drow = pl.pallas_call(
        _colsum_body,
        grid=(R,),
        in_specs=[pl.BlockSpec((_BR, N), lambda r: (r, 0))],
        out_specs=pl.BlockSpec((1, N), lambda r: (0, 0)),
        out_shape=jax.ShapeDtypeStruct((1, N), f32),
        scratch_shapes=[pltpu.VMEM((8, N), f32)],
        compiler_params=cparams,
    )(A)
    dcol = drow.reshape(N, 1)

    Lb16, h1 = pl.pallas_call(
        _layer1_body,
        grid=(R,),
        in_specs=[
            pl.BlockSpec((_BR, N), lambda r: (r, 0)),
            pl.BlockSpec((_BR, 1), lambda r: (r, 0)),
            pl.BlockSpec((1, N), lambda r: (0, 0)),
            pl.BlockSpec((N, xf.shape[1]), lambda r: (0, 0)),
            pl.BlockSpec(W0.shape, lambda r: (0, 0)),
        ],
        out_specs=[
            pl.BlockSpec((_BR, N), lambda r: (r, 0)),
            pl.BlockSpec((N, F), lambda r: (0, 0)),
        ],
        out_shape=[
            jax.ShapeDtypeStruct((N, N), bf16),
            jax.ShapeDtypeStruct((N, F), f32),
        ],
        scratch_shapes=[
            pltpu.VMEM((N, xf.shape[1]), f32),
            pltpu.VMEM((N, xf.shape[1]), bf16),
        ],
        compiler_params=cparams,
    )(A, dcol, drow, xf, W0)

    def bn_relu(h, g, b):
        hb = h.reshape(1, N, F)
        mu = jnp.mean(hb, axis=(0, 1), keepdims=True)
        var = jnp.var(hb, axis=(0, 1), keepdims=True)
        hb = g * (hb - mu) / jnp.sqrt(var + 1e-5) + b
        return jax.nn.relu(hb).reshape(N, F)

    def layerN(xk, W):
        return pl.pallas_call(
            _layerN_body,
            grid=(R,),
            in_specs=[
                pl.BlockSpec((_BR, N), lambda r: (r, 0)),
                pl.BlockSpec((N, F), lambda r: (0, 0)),
                pl.BlockSpec(W.shape, lambda r: (0, 0)),
            ],
            out_specs=pl.BlockSpec((N, F), lambda r: (0, 0)),
            out_shape=jax.ShapeDtypeStruct((N, F), f32),
            scratch_shapes=[
                pltpu.VMEM((N, F), f32),
                pltpu.VMEM((N, F), bf16),
            ],
            compiler_params=cparams,
        )(Lb16, xk, W)

    x1 = bn_relu(h1, g0, beta0)
    x2 = bn_relu(layerN(x1, W1), g1, beta1)
    x3 = bn_relu(layerN(x2, W2), g2, beta2)

    cmap, seq = pl.pallas_call(
        _decoder_body,
        grid=(R,),
        in_specs=[
            pl.BlockSpec((N, F), lambda r: (0, 0)),
            pl.BlockSpec((N, F), lambda r: (0, 0)),
            pl.BlockSpec((N, F), lambda r: (0, 0)),
            pl.BlockSpec(dec_W.shape, lambda r: (0, 0)),
            pl.BlockSpec((1, FD), lambda r: (0, 0)),
            pl.BlockSpec(seq_W.shape, lambda r: (0, 0)),
            pl.BlockSpec((1, S), lambda r: (0, 0)),
        ],
        out_specs=[
            pl.BlockSpec((_BR, N), lambda r: (r, 0)),
            pl.BlockSpec((N, S), lambda r: (0, 0)),
        ],
        out_shape=[
            jax.ShapeDtypeStruct((N, N), f32),
            jax.ShapeDtypeStruct((N, S), f32),
        ],
        scratch_shapes=[pltpu.VMEM((N, FD), f32)],
        compiler_params=cparams,
    )(x1, x2, x3, dec_W.astype(f32), dec_b.reshape(1, -1),
      seq_W, seq_b.reshape(1, -1))

    return (cmap.reshape(Bb, N, N), seq.reshape(Bb, N, S))
